# Initial kernel scaffold; baseline (speedup 1.0000x reference)
import jax
import jax.numpy as jnp
from jax.experimental import pallas as pl


def kernel(states, w1, b1, w2, b2, w3, b3):
    raise NotImplementedError("write your pallas kernel here")


# speedup vs baseline: 2.2291x; 2.2291x over previous
import functools

import jax
import jax.numpy as jnp
from jax.experimental import pallas as pl
from jax.experimental.pallas import tpu as pltpu


def _round_up(x, m):
    return ((x + m - 1) // m) * m


def _mlp_rowmajor_kernel(x_ref, w1_ref, b1r_ref, w2_ref, b2r_ref, w3r_ref,
                         b3_ref, out_ref):
    x = x_ref[...]

    h1 = jax.lax.dot_general(
        x, w1_ref[...], (((1,), (1,)), ((), ())),
        preferred_element_type=jnp.float32)
    h1 = jnp.maximum(h1 + b1r_ref[...], 0.0)

    h2 = jax.lax.dot_general(
        h1, w2_ref[...], (((1,), (1,)), ((), ())),
        preferred_element_type=jnp.float32)
    h2 = jnp.maximum(h2 + b2r_ref[...], 0.0)

    v = jnp.sum(h2 * w3r_ref[...], axis=1, keepdims=True)
    out_ref[...] = (v + b3_ref[0]).astype(out_ref.dtype)


@functools.partial(jax.jit, static_argnames=("tb",))
def _v_forward(states, w1, b1, w2, b2, w3, b3, *, tb=2048):
    states = states.astype(jnp.float32)
    B, state_dims = states.shape

    tb = _round_up(min(tb, B), 256)
    b_pad = _round_up(B, tb)
    if b_pad != B:
        states = jnp.pad(states, ((0, b_pad - B), (0, 0)))

    b1_row = b1.reshape((1, -1)).astype(jnp.float32)
    b2_row = b2.reshape((1, -1)).astype(jnp.float32)
    w3_row = w3.reshape((1, -1)).astype(jnp.float32)
    b3_s = b3.reshape((1,)).astype(jnp.float32)

    out = pl.pallas_call(
        _mlp_rowmajor_kernel,
        out_shape=jax.ShapeDtypeStruct((b_pad, 1), jnp.float32),
        grid=(b_pad // tb,),
        in_specs=[
            pl.BlockSpec((tb, state_dims), lambda i: (i, 0)),
            pl.BlockSpec(w1.shape, lambda i: (0, 0)),
            pl.BlockSpec(b1_row.shape, lambda i: (0, 0)),
            pl.BlockSpec(w2.shape, lambda i: (0, 0)),
            pl.BlockSpec(b2_row.shape, lambda i: (0, 0)),
            pl.BlockSpec(w3_row.shape, lambda i: (0, 0)),
            pl.BlockSpec(memory_space=pltpu.MemorySpace.SMEM),
        ],
        out_specs=pl.BlockSpec((tb, 1), lambda i: (i, 0)),
        compiler_params=pltpu.CompilerParams(
            dimension_semantics=("parallel",)),
    )(states, w1, b1_row, w2, b2_row, w3_row, b3_s)

    return out[:B, :]


def kernel(states, w1, b1, w2, b2, w3, b3):
    return _v_forward(states, w1, b1, w2, b2, w3, b3)


# speedup vs baseline: 2.4891x; 1.1167x over previous
import functools

import jax
import jax.numpy as jnp
from jax.experimental import pallas as pl
from jax.experimental.pallas import tpu as pltpu


def _round_up(x, m):
    return ((x + m - 1) // m) * m


def _mlp_rowmajor_kernel(x_ref, w1_ref, b1r_ref, w2_ref, b2r_ref, w3r_ref,
                         b3_ref, out_ref):
    x = x_ref[...]

    h1 = jax.lax.dot_general(
        x, w1_ref[...], (((1,), (1,)), ((), ())),
        preferred_element_type=jnp.float32)
    h1 = jnp.maximum(h1 + b1r_ref[...], 0.0)

    h2 = jax.lax.dot_general(
        h1, w2_ref[...], (((1,), (1,)), ((), ())),
        preferred_element_type=jnp.float32)
    h2 = jnp.maximum(h2 + b2r_ref[...], 0.0)

    v = jnp.sum(h2 * w3r_ref[...], axis=1, keepdims=True)
    out_ref[...] = (v + b3_ref[0]).astype(out_ref.dtype)


@functools.partial(jax.jit, static_argnames=("tb",))
def _v_forward(states, w1, b1, w2, b2, w3, b3, *, tb=4096):
    states = states.astype(jnp.float32)
    B, state_dims = states.shape

    tb = _round_up(min(tb, B), 256)
    b_pad = _round_up(B, tb)
    if b_pad != B:
        states = jnp.pad(states, ((0, b_pad - B), (0, 0)))

    b1_row = b1.reshape((1, -1)).astype(jnp.float32)
    b2_row = b2.reshape((1, -1)).astype(jnp.float32)
    w3_row = w3.reshape((1, -1)).astype(jnp.float32)
    b3_s = b3.reshape((1,)).astype(jnp.float32)

    out = pl.pallas_call(
        _mlp_rowmajor_kernel,
        out_shape=jax.ShapeDtypeStruct((b_pad, 1), jnp.float32),
        grid=(b_pad // tb,),
        in_specs=[
            pl.BlockSpec((tb, state_dims), lambda i: (i, 0)),
            pl.BlockSpec(w1.shape, lambda i: (0, 0)),
            pl.BlockSpec(b1_row.shape, lambda i: (0, 0)),
            pl.BlockSpec(w2.shape, lambda i: (0, 0)),
            pl.BlockSpec(b2_row.shape, lambda i: (0, 0)),
            pl.BlockSpec(w3_row.shape, lambda i: (0, 0)),
            pl.BlockSpec(memory_space=pltpu.MemorySpace.SMEM),
        ],
        out_specs=pl.BlockSpec((tb, 1), lambda i: (i, 0)),
        compiler_params=pltpu.CompilerParams(
            dimension_semantics=("parallel",)),
    )(states, w1, b1_row, w2, b2_row, w3_row, b3_s)

    return out[:B, :]


def kernel(states, w1, b1, w2, b2, w3, b3):
    return _v_forward(states, w1, b1, w2, b2, w3, b3)


# speedup vs baseline: 2.5490x; 1.0241x over previous
import functools

import jax
import jax.numpy as jnp
from jax.experimental import pallas as pl
from jax.experimental.pallas import tpu as pltpu


def _round_up(x, m):
    return ((x + m - 1) // m) * m


def _mlp_rowmajor_kernel(x_ref, w1_ref, b1r_ref, w2_ref, b2r_ref, w3r_ref,
                         b3_ref, out_ref):
    x = x_ref[...]

    h1 = jax.lax.dot_general(
        x, w1_ref[...], (((1,), (1,)), ((), ())),
        preferred_element_type=jnp.float32)
    h1 = jnp.maximum(h1 + b1r_ref[...], 0.0)

    h2 = jax.lax.dot_general(
        h1, w2_ref[...], (((1,), (1,)), ((), ())),
        preferred_element_type=jnp.float32)
    h2 = jnp.maximum(h2 + b2r_ref[...], 0.0)

    v = jnp.sum(h2 * w3r_ref[...], axis=1, keepdims=True)
    out_ref[...] = (v + b3_ref[0]).astype(out_ref.dtype)


@functools.partial(jax.jit, static_argnames=("tb",))
def _v_forward(states, w1, b1, w2, b2, w3, b3, *, tb=8192):
    states = states.astype(jnp.float32)
    B, state_dims = states.shape

    tb = _round_up(min(tb, B), 256)
    b_pad = _round_up(B, tb)
    if b_pad != B:
        states = jnp.pad(states, ((0, b_pad - B), (0, 0)))

    b1_row = b1.reshape((1, -1)).astype(jnp.float32)
    b2_row = b2.reshape((1, -1)).astype(jnp.float32)
    w3_row = w3.reshape((1, -1)).astype(jnp.float32)
    b3_s = b3.reshape((1,)).astype(jnp.float32)

    out = pl.pallas_call(
        _mlp_rowmajor_kernel,
        out_shape=jax.ShapeDtypeStruct((b_pad, 1), jnp.float32),
        grid=(b_pad // tb,),
        in_specs=[
            pl.BlockSpec((tb, state_dims), lambda i: (i, 0)),
            pl.BlockSpec(w1.shape, lambda i: (0, 0)),
            pl.BlockSpec(b1_row.shape, lambda i: (0, 0)),
            pl.BlockSpec(w2.shape, lambda i: (0, 0)),
            pl.BlockSpec(b2_row.shape, lambda i: (0, 0)),
            pl.BlockSpec(w3_row.shape, lambda i: (0, 0)),
            pl.BlockSpec(memory_space=pltpu.MemorySpace.SMEM),
        ],
        out_specs=pl.BlockSpec((tb, 1), lambda i: (i, 0)),
        compiler_params=pltpu.CompilerParams(
            dimension_semantics=("parallel",)),
    )(states, w1, b1_row, w2, b2_row, w3_row, b3_s)

    return out[:B, :]


def kernel(states, w1, b1, w2, b2, w3, b3):
    return _v_forward(states, w1, b1, w2, b2, w3, b3)
